# Initial kernel scaffold; baseline (speedup 1.0000x reference)
import jax
import jax.numpy as jnp
from jax.experimental import pallas as pl


def kernel(x, edge_index, W1l, b1l, W1r, W2l, b2l, W2r):
    raise NotImplementedError("write your pallas kernel here")


# speedup vs baseline: 4.1250x; 4.1250x over previous
import functools

import jax
import jax.numpy as jnp
from jax import lax
from jax.experimental import pallas as pl
from jax.experimental.pallas import tpu as pltpu
from jax.experimental.pallas import tpu_sc as plsc

N = 10000
E = 320000
D = 128
NP = 10112

NC = 2
NS = 16
NW = NC * NS
CHUNK = 128
EPW = 10112
NCH = EPW // CHUNK
EPAD = EPW * NW
RPS = NP // NS

_mesh = plsc.VectorSubcoreMesh(core_axis_name="c", subcore_axis_name="s")


@functools.partial(
    pl.kernel,
    out_type=jax.ShapeDtypeStruct((NC, NP, D), jnp.float32),
    mesh=_mesh,
    scratch_types=[
        pltpu.VMEM((CHUNK,), jnp.int32),
        pltpu.VMEM((CHUNK,), jnp.int32),
        pltpu.VMEM((CHUNK, D), jnp.float32),
        pltpu.VMEM_SHARED((NP, D), jnp.float32),
        pltpu.SemaphoreType.DMA,
    ],
)
def _agg(x_hbm, src_hbm, dst_hbm, z_hbm, out_hbm, src_v, dst_v, rows_v,
         acc_sh, sem):
    c = lax.axis_index("c")
    s = lax.axis_index("s")
    wid = s * NC + c

    pltpu.sync_copy(z_hbm.at[pl.ds(s * RPS, RPS)],
                    acc_sh.at[pl.ds(s * RPS, RPS)])
    plsc.subcore_barrier()

    @pl.loop(0, NCH)
    def _(j):
        pltpu.sync_copy(src_hbm.at[wid, j], src_v)
        pltpu.sync_copy(dst_hbm.at[wid, j], dst_v)
        pltpu.async_copy(x_hbm.at[src_v], rows_v, sem).wait()
        pltpu.sync_copy(rows_v, acc_sh.at[dst_v], add=True)

    plsc.subcore_barrier()
    pltpu.sync_copy(acc_sh.at[pl.ds(s * RPS, RPS)],
                    out_hbm.at[c, pl.ds(s * RPS, RPS)])


@functools.partial(
    pl.kernel,
    out_type=jax.ShapeDtypeStruct((NC, NP, D), jnp.float32),
    mesh=_mesh,
    scratch_types=[
        pltpu.VMEM((CHUNK,), jnp.int32),
        pltpu.VMEM((CHUNK, D), jnp.float32),
        pltpu.VMEM_SHARED((NP, D), jnp.float32),
        pltpu.SemaphoreType.DMA,
    ],
)
def _count(ones_hbm, dst_hbm, z_hbm, out_hbm, dst_v, ones_v, acc_sh, sem):
    c = lax.axis_index("c")
    s = lax.axis_index("s")
    wid = s * NC + c

    pltpu.sync_copy(z_hbm.at[pl.ds(s * RPS, RPS)],
                    acc_sh.at[pl.ds(s * RPS, RPS)])
    pltpu.sync_copy(ones_hbm, ones_v)
    plsc.subcore_barrier()

    @pl.loop(0, NCH)
    def _(j):
        pltpu.sync_copy(dst_hbm.at[wid, j], dst_v)
        pltpu.sync_copy(ones_v, acc_sh.at[dst_v], add=True)

    plsc.subcore_barrier()
    pltpu.sync_copy(acc_sh.at[pl.ds(s * RPS, RPS)],
                    out_hbm.at[c, pl.ds(s * RPS, RPS)])


BT = 1264


def _dot_t(a, w):
    return lax.dot_general(a, w, (((1,), (1,)), ((), ())),
                           precision=lax.Precision.HIGHEST,
                           preferred_element_type=jnp.float32)


def _tc_layer1(acc_ref, cnt_ref, x_ref, wl_ref, b_ref, wr_ref, h_ref,
               invc_ref):
    ssum = acc_ref[0] + acc_ref[1]
    cnt = cnt_ref[0, :, :1] + cnt_ref[1, :, :1]
    invc = 1.0 / jnp.maximum(cnt, 1.0)
    agg = ssum * invc
    out = _dot_t(agg, wl_ref[...]) + _dot_t(x_ref[...], wr_ref[...])
    out = out + b_ref[...]
    h = jnp.maximum(out, 0.0)
    rows = pl.program_id(0) * BT + lax.broadcasted_iota(jnp.int32, (BT, 1), 0)
    h_ref[...] = jnp.where(rows < N, h, 0.0)
    invc_ref[...] = jnp.broadcast_to(invc, (BT, D))


def _tc_layer2(acc_ref, h_ref, invc_ref, wl_ref, b_ref, wr_ref, o_ref):
    ssum = acc_ref[0] + acc_ref[1]
    agg = ssum * invc_ref[...]
    o_ref[...] = _dot_t(agg, wl_ref[...]) + _dot_t(h_ref[...], wr_ref[...]) \
        + b_ref[...]


_spec_rows = pl.BlockSpec((BT, D), lambda i: (i, 0))
_spec_acc = pl.BlockSpec((NC, BT, D), lambda i: (0, i, 0))
_spec_w = pl.BlockSpec((D, D), lambda i: (0, 0))
_spec_b = pl.BlockSpec((1, D), lambda i: (0, 0))


@jax.jit
def kernel(x, edge_index, W1l, b1l, W1r, W2l, b2l, W2r):
    src = edge_index[0].astype(jnp.int32)
    dst = edge_index[1].astype(jnp.int32)
    pad = jnp.full((EPAD - E,), N, jnp.int32)
    src_r = jnp.concatenate([src, pad]).reshape(NW, NCH, CHUNK)
    dst_r = jnp.concatenate([dst, pad]).reshape(NW, NCH, CHUNK)

    xp = jnp.pad(x, ((0, NP - N), (0, 0)))
    z_d = jnp.zeros((NP, D), jnp.float32)
    ones = jnp.ones((CHUNK, D), jnp.float32)

    cnt = _count(ones, dst_r, z_d)
    acc1 = _agg(xp, src_r, dst_r, z_d)

    h, invc = pl.pallas_call(
        _tc_layer1,
        grid=(NP // BT,),
        in_specs=[_spec_acc, _spec_acc, _spec_rows, _spec_w, _spec_b, _spec_w],
        out_specs=(_spec_rows, _spec_rows),
        out_shape=(jax.ShapeDtypeStruct((NP, D), jnp.float32),
                   jax.ShapeDtypeStruct((NP, D), jnp.float32)),
    )(acc1, cnt, xp, W1l, b1l.reshape(1, D), W1r)

    acc2 = _agg(h, src_r, dst_r, z_d)

    out = pl.pallas_call(
        _tc_layer2,
        grid=(NP // BT,),
        in_specs=[_spec_acc, _spec_rows, _spec_rows, _spec_w, _spec_b,
                  _spec_w],
        out_specs=_spec_rows,
        out_shape=jax.ShapeDtypeStruct((NP, D), jnp.float32),
    )(acc2, h, invc, W2l, b2l.reshape(1, D), W2r)

    return out[:N]
